# baseline (speedup 1.0000x reference)
import functools
import jax
import jax.numpy as jnp
from jax import lax
from jax.experimental import pallas as pl
from jax.experimental.pallas import tpu as pltpu
from jax.experimental.pallas import tpu_sc as plsc

N = 10000
NPAD = 10240
E = 160000
B = 128
NBATCH = 80
EPAD = 16 * NBATCH * B
F = 128
ROWS = NPAD // 16
RCH = ROWS // B
BG = 64
SG = 8
NSG = 20
KSTEPS = 10
ALPHA = 0.3
RB = 1280


def _deg_body(dst_hbm, deg_out, idx_v, buf_v, deg_sp):
    c = lax.axis_index("c")
    s = lax.axis_index("s")
    base = s * ROWS

    pltpu.sync_copy(dst_hbm.at[s], idx_v)

    def fill_buf(i, _):
        for j in range(F // 16):
            buf_v[i, pl.ds(j * 16, 16)] = jnp.ones((16,), jnp.float32)
        return 0
    lax.fori_loop(0, B, fill_buf, 0)

    def init_chunk(q, _):
        pltpu.sync_copy(buf_v, deg_sp.at[pl.ds(base + q * B, B)])
        return 0
    lax.fori_loop(0, RCH, init_chunk, 0)
    plsc.subcore_barrier()

    def scat(j, _):
        pltpu.sync_copy(buf_v, deg_sp.at[idx_v.at[j]], add=True)
        return 0
    lax.fori_loop(0, NBATCH, scat, 0)
    plsc.subcore_barrier()

    @pl.when(c == 0)
    def _():
        def out_chunk(q, _):
            pltpu.sync_copy(deg_sp.at[pl.ds(base + q * B, B)], buf_v)
            pltpu.sync_copy(buf_v, deg_out.at[pl.ds(base + q * B, B)])
            return 0
        lax.fori_loop(0, RCH, out_chunk, 0)


def _prop_body(src_hbm, dst_hbm, u0_hbm, c1_hbm, uk_hbm,
               sia, dia, sib, dib, gb0, gb1, gb2, gb3,
               sem0, sem1, sem2, sem3, sema, semb, acc_sp):
    c = lax.axis_index("c")
    s = lax.axis_index("s")
    base = s * ROWS

    def init_chunk(q, _):
        r = base + q * BG
        pltpu.sync_copy(u0_hbm.at[c, pl.ds(r, BG)], gb0)
        pltpu.sync_copy(gb0, uk_hbm.at[c, pl.ds(r, BG)])
        pltpu.sync_copy(gb0, acc_sp.at[pl.ds(r, BG)])
        return 0
    lax.fori_loop(0, ROWS // BG, init_chunk, 0)
    plsc.subcore_barrier()

    gbufs = (gb0, gb1, gb2, gb3)
    gsems = (sem0, sem1, sem2, sem3)

    def kstep(_, carry):
        pltpu.sync_copy(src_hbm.at[s, 0], sia)
        pltpu.sync_copy(dst_hbm.at[s, 0], dia)
        pltpu.sync_copy(src_hbm.at[s, 1], sib)
        pltpu.sync_copy(dst_hbm.at[s, 1], dib)
        for b in range(4):
            pltpu.async_copy(uk_hbm.at[c].at[sia.at[b]], gbufs[b], gsems[b])

        def gpair(go, _):
            for p in range(2):
                sP, dP = (sia, dia) if p == 0 else (sib, dib)
                sQ = sib if p == 0 else sia
                semQ = semb if p == 0 else sema
                g = 2 * go + p
                for b in range(8):
                    sl = b % 4
                    pltpu.make_async_copy(
                        uk_hbm.at[c].at[sP.at[b]], gbufs[sl], gsems[sl]).wait()
                    pltpu.sync_copy(gbufs[sl], acc_sp.at[dP.at[b]], add=True)
                    if b < 4:
                        pltpu.async_copy(
                            uk_hbm.at[c].at[sP.at[b + 4]], gbufs[sl], gsems[sl])
                    else:
                        if b == 4:
                            @pl.when(jnp.logical_and(g >= 1, g + 1 < NSG))
                            def _(sQ=sQ, semQ=semQ):
                                pltpu.make_async_copy(
                                    src_hbm.at[s, 0], sQ, semQ).wait()
                                pltpu.make_async_copy(
                                    src_hbm.at[s, 0], sQ, semQ).wait()

                        @pl.when(g + 1 < NSG)
                        def _(b=b, sl=sl, sQ=sQ):
                            pltpu.async_copy(
                                uk_hbm.at[c].at[sQ.at[b - 4]], gbufs[sl], gsems[sl])

                @pl.when(g + 2 < NSG)
                def _(g=g, sP=sP, dP=dP, semP=(sema if p == 0 else semb)):
                    pltpu.async_copy(src_hbm.at[s, g + 2], sP, semP)
                    pltpu.async_copy(dst_hbm.at[s, g + 2], dP, semP)
            return 0
        lax.fori_loop(0, NSG // 2, gpair, 0)
        plsc.subcore_barrier()

        NCH = ROWS // BG
        abufs = (gb0, gb3)
        ds_u = [None] * NCH
        ds_a = [None] * NCH
        l0 = pltpu.async_copy(acc_sp.at[pl.ds(base, BG)], gb0, sem0)
        l1 = pltpu.async_copy(u0_hbm.at[c, pl.ds(base, BG)], gb1, sem1)
        l2 = pltpu.async_copy(c1_hbm.at[c, pl.ds(base, BG)], gb2, sem2)
        for q in range(NCH):
            ab = abufs[q % 2]
            r = base + q * BG
            l0.wait()
            l1.wait()
            l2.wait()

            @plsc.parallel_loop(0, BG, unroll=4)
            def _(i, ab=ab):
                for j in range(F // 16):
                    sl = (i, pl.ds(j * 16, 16))
                    ab[sl] = gb2[sl] * ab[sl] + ALPHA * gb1[sl]

            ds_u[q] = pltpu.async_copy(ab, uk_hbm.at[c, pl.ds(r, BG)], semb)
            ds_a[q] = pltpu.async_copy(ab, acc_sp.at[pl.ds(r, BG)], sema)
            if q + 1 < NCH:
                rn = base + (q + 1) * BG
                nab = abufs[(q + 1) % 2]
                if q >= 1:
                    ds_u[q - 1].wait()
                    ds_a[q - 1].wait()
                l0 = pltpu.async_copy(acc_sp.at[pl.ds(rn, BG)], nab, sem0)
                l1 = pltpu.async_copy(u0_hbm.at[c, pl.ds(rn, BG)], gb1, sem1)
                l2 = pltpu.async_copy(c1_hbm.at[c, pl.ds(rn, BG)], gb2, sem2)
        ds_u[NCH - 2].wait()
        ds_a[NCH - 2].wait()
        ds_u[NCH - 1].wait()
        ds_a[NCH - 1].wait()
        plsc.subcore_barrier()
        return carry
    lax.fori_loop(0, KSTEPS, kstep, 0)


def _prep_body(x_ref, w1_ref, b1_ref, w2_ref, b2_ref, deg_ref, u0_ref, c1f_ref):
    h1 = jnp.maximum(
        jnp.dot(x_ref[...], w1_ref[...], preferred_element_type=jnp.float32)
        + b1_ref[...], 0.0)
    h = jnp.dot(h1, w2_ref[...], preferred_element_type=jnp.float32) + b2_ref[...]
    deg = deg_ref[...][:, 0]
    dinv = lax.rsqrt(deg)
    u0 = h * dinv[:, None]
    u0_ref[0] = u0[:, :F]
    u0_ref[1] = u0[:, F:]
    c1 = (1.0 - ALPHA) / deg
    c1b = jnp.broadcast_to(c1[:, None], (RB, F))
    c1f_ref[0] = c1b
    c1f_ref[1] = c1b


def _out_body(uk_ref, deg_ref, o_ref):
    deg = deg_ref[...][:, 0]
    z = jnp.concatenate([uk_ref[0], uk_ref[1]], axis=1) * jnp.sqrt(deg)[:, None]
    m = jnp.max(z, axis=1, keepdims=True)
    e = jnp.exp(z - m)
    lse = jnp.log(jnp.sum(e, axis=1, keepdims=True)) + m
    o_ref[...] = z - lse


_sc_mesh = plsc.VectorSubcoreMesh(core_axis_name="c", subcore_axis_name="s")

_deg_call = functools.partial(
    pl.kernel,
    out_type=jax.ShapeDtypeStruct((NPAD, F), jnp.float32),
    mesh=_sc_mesh,
    scratch_types=[
        pltpu.VMEM((NBATCH, B), jnp.int32),
        pltpu.VMEM((B, F), jnp.float32),
        pltpu.VMEM_SHARED((NPAD, F), jnp.float32),
    ],
)(_deg_body)

_prop_call = functools.partial(
    pl.kernel,
    out_type=jax.ShapeDtypeStruct((2, NPAD, F), jnp.float32),
    mesh=_sc_mesh,
    scratch_types=[
        pltpu.VMEM((SG, BG), jnp.int32),
        pltpu.VMEM((SG, BG), jnp.int32),
        pltpu.VMEM((SG, BG), jnp.int32),
        pltpu.VMEM((SG, BG), jnp.int32),
        pltpu.VMEM((BG, F), jnp.float32),
        pltpu.VMEM((BG, F), jnp.float32),
        pltpu.VMEM((BG, F), jnp.float32),
        pltpu.VMEM((BG, F), jnp.float32),
        pltpu.SemaphoreType.DMA,
        pltpu.SemaphoreType.DMA,
        pltpu.SemaphoreType.DMA,
        pltpu.SemaphoreType.DMA,
        pltpu.SemaphoreType.DMA,
        pltpu.SemaphoreType.DMA,
        pltpu.VMEM_SHARED((NPAD, F), jnp.float32),
    ],
)(_prop_body)

_prep_call = pl.pallas_call(
    _prep_body,
    grid=(NPAD // RB,),
    in_specs=[
        pl.BlockSpec((RB, 256), lambda i: (i, 0)),
        pl.BlockSpec((256, 512), lambda i: (0, 0)),
        pl.BlockSpec((1, 512), lambda i: (0, 0)),
        pl.BlockSpec((512, 256), lambda i: (0, 0)),
        pl.BlockSpec((1, 256), lambda i: (0, 0)),
        pl.BlockSpec((RB, F), lambda i: (i, 0)),
    ],
    out_specs=[
        pl.BlockSpec((2, RB, F), lambda i: (0, i, 0)),
        pl.BlockSpec((2, RB, F), lambda i: (0, i, 0)),
    ],
    out_shape=[
        jax.ShapeDtypeStruct((2, NPAD, F), jnp.float32),
        jax.ShapeDtypeStruct((2, NPAD, F), jnp.float32),
    ],
)

_out_call = pl.pallas_call(
    _out_body,
    grid=(NPAD // RB,),
    in_specs=[
        pl.BlockSpec((2, RB, F), lambda i: (0, i, 0)),
        pl.BlockSpec((RB, F), lambda i: (i, 0)),
    ],
    out_specs=pl.BlockSpec((RB, 256), lambda i: (i, 0)),
    out_shape=jax.ShapeDtypeStruct((NPAD, 256), jnp.float32),
)


@jax.jit
def kernel(x, edge_index, W1, b1, W2, b2):
    src = edge_index[0].astype(jnp.int32)
    dst = edge_index[1].astype(jnp.int32)
    pad = EPAD - E
    src3 = jnp.concatenate([src, jnp.zeros((pad,), jnp.int32)]).reshape(16, NBATCH, B)
    dst3 = jnp.concatenate([dst, jnp.full((pad,), N, jnp.int32)]).reshape(16, NBATCH, B)
    x_pad = jnp.pad(x, ((0, NPAD - N), (0, 0)))

    deg = _deg_call(dst3)
    u0, c1f = _prep_call(x_pad, W1, b1.reshape(1, -1), W2, b2.reshape(1, -1), deg)
    uk = _prop_call(src3.reshape(16, NSG, SG, BG), dst3.reshape(16, NSG, SG, BG),
                    u0, c1f)
    out = _out_call(uk, deg)
    return out[:N]


# speedup vs baseline: 1.0421x; 1.0421x over previous
import functools
import jax
import jax.numpy as jnp
from jax import lax
from jax.experimental import pallas as pl
from jax.experimental.pallas import tpu as pltpu
from jax.experimental.pallas import tpu_sc as plsc

N = 10000
NPAD = 10240
E = 160000
B = 128
NBATCH = 80
EPAD = 16 * NBATCH * B
F = 128
ROWS = NPAD // 16
RCH = ROWS // B
BG = 128
SG = 4
NSG = 20
NSLOT = 2
UH = 64
KSTEPS = 10
ALPHA = 0.3
RB = 1280


def _deg_body(dst_hbm, deg_out, idx_v, buf_v, deg_sp):
    c = lax.axis_index("c")
    s = lax.axis_index("s")
    base = s * ROWS

    pltpu.sync_copy(dst_hbm.at[s], idx_v)

    def fill_buf(i, _):
        for j in range(F // 16):
            buf_v[i, pl.ds(j * 16, 16)] = jnp.ones((16,), jnp.float32)
        return 0
    lax.fori_loop(0, B, fill_buf, 0)

    def init_chunk(q, _):
        pltpu.sync_copy(buf_v, deg_sp.at[pl.ds(base + q * B, B)])
        return 0
    lax.fori_loop(0, RCH, init_chunk, 0)
    plsc.subcore_barrier()

    def scat(j, _):
        pltpu.sync_copy(buf_v, deg_sp.at[idx_v.at[j]], add=True)
        return 0
    lax.fori_loop(0, NBATCH, scat, 0)
    plsc.subcore_barrier()

    @pl.when(c == 0)
    def _():
        def out_chunk(q, _):
            pltpu.sync_copy(deg_sp.at[pl.ds(base + q * B, B)], buf_v)
            pltpu.sync_copy(buf_v, deg_out.at[pl.ds(base + q * B, B)])
            return 0
        lax.fori_loop(0, RCH, out_chunk, 0)


def _prop_body(src_hbm, dst_hbm, u0_hbm, c1_hbm, uk_hbm,
               sia, dia, sib, dib, gb0, gb1,
               sem0, sem1, sem2, sema, semb, acc_sp):
    c = lax.axis_index("c")
    s = lax.axis_index("s")
    base = s * ROWS

    def init_chunk(q, _):
        r = base + q * BG
        pltpu.sync_copy(u0_hbm.at[c, pl.ds(r, BG)], gb0)
        pltpu.sync_copy(gb0, uk_hbm.at[c, pl.ds(r, BG)])
        pltpu.sync_copy(gb0, acc_sp.at[pl.ds(r, BG)])
        return 0
    lax.fori_loop(0, ROWS // BG, init_chunk, 0)
    plsc.subcore_barrier()

    gbufs = (gb0, gb1)
    gsems = (sem0, sem1)

    def kstep(_, carry):
        pltpu.sync_copy(src_hbm.at[s, 0], sia)
        pltpu.sync_copy(dst_hbm.at[s, 0], dia)
        pltpu.sync_copy(src_hbm.at[s, 1], sib)
        pltpu.sync_copy(dst_hbm.at[s, 1], dib)
        for b in range(NSLOT):
            pltpu.async_copy(uk_hbm.at[c].at[sia.at[b]], gbufs[b], gsems[b])

        def gpair(go, _):
            for p in range(2):
                sP, dP = (sia, dia) if p == 0 else (sib, dib)
                sQ = sib if p == 0 else sia
                semQ = semb if p == 0 else sema
                g = 2 * go + p
                for b in range(SG):
                    sl = b % NSLOT
                    pltpu.make_async_copy(
                        uk_hbm.at[c].at[sP.at[b]], gbufs[sl], gsems[sl]).wait()
                    pltpu.sync_copy(gbufs[sl], acc_sp.at[dP.at[b]], add=True)
                    if b < SG - NSLOT:
                        pltpu.async_copy(
                            uk_hbm.at[c].at[sP.at[b + NSLOT]], gbufs[sl], gsems[sl])
                    else:
                        if b == SG - NSLOT:
                            @pl.when(jnp.logical_and(g >= 1, g + 1 < NSG))
                            def _(sQ=sQ, semQ=semQ):
                                pltpu.make_async_copy(
                                    src_hbm.at[s, 0], sQ, semQ).wait()
                                pltpu.make_async_copy(
                                    src_hbm.at[s, 0], sQ, semQ).wait()

                        @pl.when(g + 1 < NSG)
                        def _(b=b, sl=sl, sQ=sQ):
                            pltpu.async_copy(
                                uk_hbm.at[c].at[sQ.at[b - (SG - NSLOT)]],
                                gbufs[sl], gsems[sl])

                @pl.when(g + 2 < NSG)
                def _(g=g, sP=sP, dP=dP, semP=(sema if p == 0 else semb)):
                    pltpu.async_copy(src_hbm.at[s, g + 2], sP, semP)
                    pltpu.async_copy(dst_hbm.at[s, g + 2], dP, semP)
            return 0
        lax.fori_loop(0, NSG // 2, gpair, 0)
        plsc.subcore_barrier()

        NCH = ROWS // UH
        abufs = (gb0.at[pl.ds(0, UH)], gb0.at[pl.ds(UH, UH)])
        u0v = gb1.at[pl.ds(0, UH)]
        c1v = gb1.at[pl.ds(UH, UH)]
        ds_u = [None] * NCH
        ds_a = [None] * NCH
        l0 = pltpu.async_copy(acc_sp.at[pl.ds(base, UH)], abufs[0], sem0)
        l1 = pltpu.async_copy(u0_hbm.at[c, pl.ds(base, UH)], u0v, sem1)
        l2 = pltpu.async_copy(c1_hbm.at[c, pl.ds(base, UH)], c1v, sem2)
        for q in range(NCH):
            ab = abufs[q % 2]
            r = base + q * UH
            l0.wait()
            l1.wait()
            l2.wait()

            def row(i, _, ab=ab):
                for j in range(F // 16):
                    sl = (i, pl.ds(j * 16, 16))
                    ab[sl] = c1v[sl] * ab[sl] + ALPHA * u0v[sl]
                return 0
            lax.fori_loop(0, UH, row, 0)

            ds_u[q] = pltpu.async_copy(ab, uk_hbm.at[c, pl.ds(r, UH)], semb)
            ds_a[q] = pltpu.async_copy(ab, acc_sp.at[pl.ds(r, UH)], sema)
            if q + 1 < NCH:
                rn = base + (q + 1) * UH
                nab = abufs[(q + 1) % 2]
                if q >= 1:
                    ds_u[q - 1].wait()
                    ds_a[q - 1].wait()
                l0 = pltpu.async_copy(acc_sp.at[pl.ds(rn, UH)], nab, sem0)
                l1 = pltpu.async_copy(u0_hbm.at[c, pl.ds(rn, UH)], u0v, sem1)
                l2 = pltpu.async_copy(c1_hbm.at[c, pl.ds(rn, UH)], c1v, sem2)
        ds_u[NCH - 2].wait()
        ds_a[NCH - 2].wait()
        ds_u[NCH - 1].wait()
        ds_a[NCH - 1].wait()
        plsc.subcore_barrier()
        return carry
    lax.fori_loop(0, KSTEPS, kstep, 0)


def _prep_body(x_ref, w1_ref, b1_ref, w2_ref, b2_ref, deg_ref, u0_ref, c1f_ref):
    h1 = jnp.maximum(
        jnp.dot(x_ref[...], w1_ref[...], preferred_element_type=jnp.float32)
        + b1_ref[...], 0.0)
    h = jnp.dot(h1, w2_ref[...], preferred_element_type=jnp.float32) + b2_ref[...]
    deg = deg_ref[...][:, 0]
    dinv = lax.rsqrt(deg)
    u0 = h * dinv[:, None]
    u0_ref[0] = u0[:, :F]
    u0_ref[1] = u0[:, F:]
    c1 = (1.0 - ALPHA) / deg
    c1b = jnp.broadcast_to(c1[:, None], (RB, F))
    c1f_ref[0] = c1b
    c1f_ref[1] = c1b


def _out_body(uk_ref, deg_ref, o_ref):
    deg = deg_ref[...][:, 0]
    z = jnp.concatenate([uk_ref[0], uk_ref[1]], axis=1) * jnp.sqrt(deg)[:, None]
    m = jnp.max(z, axis=1, keepdims=True)
    e = jnp.exp(z - m)
    lse = jnp.log(jnp.sum(e, axis=1, keepdims=True)) + m
    o_ref[...] = z - lse


_sc_mesh = plsc.VectorSubcoreMesh(core_axis_name="c", subcore_axis_name="s")

_deg_call = functools.partial(
    pl.kernel,
    out_type=jax.ShapeDtypeStruct((NPAD, F), jnp.float32),
    mesh=_sc_mesh,
    scratch_types=[
        pltpu.VMEM((NBATCH, B), jnp.int32),
        pltpu.VMEM((B, F), jnp.float32),
        pltpu.VMEM_SHARED((NPAD, F), jnp.float32),
    ],
)(_deg_body)

_prop_call = functools.partial(
    pl.kernel,
    out_type=jax.ShapeDtypeStruct((2, NPAD, F), jnp.float32),
    mesh=_sc_mesh,
    scratch_types=[
        pltpu.VMEM((SG, BG), jnp.int32),
        pltpu.VMEM((SG, BG), jnp.int32),
        pltpu.VMEM((SG, BG), jnp.int32),
        pltpu.VMEM((SG, BG), jnp.int32),
        pltpu.VMEM((BG, F), jnp.float32),
        pltpu.VMEM((BG, F), jnp.float32),
        pltpu.SemaphoreType.DMA,
        pltpu.SemaphoreType.DMA,
        pltpu.SemaphoreType.DMA,
        pltpu.SemaphoreType.DMA,
        pltpu.SemaphoreType.DMA,
        pltpu.VMEM_SHARED((NPAD, F), jnp.float32),
    ],
)(_prop_body)

_prep_call = pl.pallas_call(
    _prep_body,
    grid=(NPAD // RB,),
    in_specs=[
        pl.BlockSpec((RB, 256), lambda i: (i, 0)),
        pl.BlockSpec((256, 512), lambda i: (0, 0)),
        pl.BlockSpec((1, 512), lambda i: (0, 0)),
        pl.BlockSpec((512, 256), lambda i: (0, 0)),
        pl.BlockSpec((1, 256), lambda i: (0, 0)),
        pl.BlockSpec((RB, F), lambda i: (i, 0)),
    ],
    out_specs=[
        pl.BlockSpec((2, RB, F), lambda i: (0, i, 0)),
        pl.BlockSpec((2, RB, F), lambda i: (0, i, 0)),
    ],
    out_shape=[
        jax.ShapeDtypeStruct((2, NPAD, F), jnp.float32),
        jax.ShapeDtypeStruct((2, NPAD, F), jnp.float32),
    ],
)

_out_call = pl.pallas_call(
    _out_body,
    grid=(NPAD // RB,),
    in_specs=[
        pl.BlockSpec((2, RB, F), lambda i: (0, i, 0)),
        pl.BlockSpec((RB, F), lambda i: (i, 0)),
    ],
    out_specs=pl.BlockSpec((RB, 256), lambda i: (i, 0)),
    out_shape=jax.ShapeDtypeStruct((NPAD, 256), jnp.float32),
)


@jax.jit
def kernel(x, edge_index, W1, b1, W2, b2):
    src = edge_index[0].astype(jnp.int32)
    dst = edge_index[1].astype(jnp.int32)
    pad = EPAD - E
    src3 = jnp.concatenate([src, jnp.zeros((pad,), jnp.int32)]).reshape(16, NBATCH, B)
    dst3 = jnp.concatenate([dst, jnp.full((pad,), N, jnp.int32)]).reshape(16, NBATCH, B)
    x_pad = jnp.pad(x, ((0, NPAD - N), (0, 0)))

    deg = _deg_call(dst3)
    u0, c1f = _prep_call(x_pad, W1, b1.reshape(1, -1), W2, b2.reshape(1, -1), deg)
    uk = _prop_call(src3.reshape(16, NSG, SG, BG), dst3.reshape(16, NSG, SG, BG),
                    u0, c1f)
    out = _out_call(uk, deg)
    return out[:N]
